# Initial kernel scaffold; baseline (speedup 1.0000x reference)
import jax
import jax.numpy as jnp
from jax.experimental import pallas as pl


def kernel(token_ids, weight):
    raise NotImplementedError("write your pallas kernel here")


# speedup vs baseline: 1.1600x; 1.1600x over previous
import functools

import jax
import jax.numpy as jnp
from jax import lax
from jax.experimental import pallas as pl
from jax.experimental.pallas import tpu as pltpu
from jax.experimental.pallas import tpu_sc as plsc

NUM_EMBEDDINGS = 100000
EMBEDDING_DIM = 128
BATCH = 4096
N_FIELDS = 26

_NC = 2
_NS = 16
_NW = _NC * _NS

_B = BATCH * N_FIELDS
_CHUNK = 128
_ROWS_PER_W = _B // _NW
_CHUNKS_PER_W = _ROWS_PER_W // _CHUNK


@functools.partial(
    pl.kernel,
    out_type=jax.ShapeDtypeStruct((_B, EMBEDDING_DIM), jnp.float32),
    mesh=plsc.VectorSubcoreMesh(core_axis_name="c", subcore_axis_name="s"),
    scratch_types=[
        pltpu.VMEM((_CHUNKS_PER_W, _CHUNK), jnp.int32),
        pltpu.VMEM((_CHUNK, EMBEDDING_DIM), jnp.float32),
        pltpu.SemaphoreType.DMA,
    ],
)
def _gather_kernel(idx_hbm, table_hbm, out_hbm, idx_v, rows_v, sem):
    wid = lax.axis_index("s") * _NC + lax.axis_index("c")
    pltpu.sync_copy(idx_hbm.at[wid], idx_v)
    row_base = wid * _ROWS_PER_W

    def body(j, _):
        pltpu.async_copy(table_hbm.at[idx_v.at[j]], rows_v, sem).wait()
        pltpu.sync_copy(rows_v, out_hbm.at[pl.ds(row_base + j * _CHUNK, _CHUNK)])
        return 0

    lax.fori_loop(0, _CHUNKS_PER_W, body, 0)


def kernel(token_ids, weight):
    idx = token_ids.reshape(_NW, _CHUNKS_PER_W, _CHUNK).astype(jnp.int32)
    out = _gather_kernel(idx, weight)
    return out.reshape(BATCH, N_FIELDS, EMBEDDING_DIM)


# speedup vs baseline: 1.2851x; 1.1078x over previous
import functools

import jax
import jax.numpy as jnp
from jax import lax
from jax.experimental import pallas as pl
from jax.experimental.pallas import tpu as pltpu
from jax.experimental.pallas import tpu_sc as plsc

NUM_EMBEDDINGS = 100000
EMBEDDING_DIM = 128
BATCH = 4096
N_FIELDS = 26

_NC = 2
_NS = 16
_NW = _NC * _NS

_B = BATCH * N_FIELDS
_CHUNK = 128
_ROWS_PER_W = _B // _NW
_CHUNKS_PER_W = _ROWS_PER_W // _CHUNK
_NBUF = 2


@functools.partial(
    pl.kernel,
    out_type=jax.ShapeDtypeStruct((_B, EMBEDDING_DIM), jnp.float32),
    mesh=plsc.VectorSubcoreMesh(core_axis_name="c", subcore_axis_name="s"),
    scratch_types=[
        pltpu.VMEM((_CHUNKS_PER_W, _CHUNK), jnp.int32),
        pltpu.VMEM((_NBUF, _CHUNK, EMBEDDING_DIM), jnp.float32),
        pltpu.SemaphoreType.DMA((_NBUF,)),
        pltpu.SemaphoreType.DMA((_NBUF,)),
    ],
)
def _gather_kernel(idx_hbm, table_hbm, out_hbm, idx_v, bufs, gsems, ssems):
    wid = lax.axis_index("s") * _NC + lax.axis_index("c")
    pltpu.sync_copy(idx_hbm.at[wid], idx_v)
    row_base = wid * _ROWS_PER_W

    def g_start(j, b):
        pltpu.async_copy(table_hbm.at[idx_v.at[j]], bufs.at[b], gsems.at[b])

    def g_wait(j, b):
        pltpu.make_async_copy(
            table_hbm.at[idx_v.at[j]], bufs.at[b], gsems.at[b]).wait()

    def out_slot(j):
        return out_hbm.at[pl.ds(row_base + j * _CHUNK, _CHUNK)]

    def s_start(j, b):
        pltpu.async_copy(bufs.at[b], out_slot(j), ssems.at[b])

    def s_wait(j, b):
        pltpu.make_async_copy(bufs.at[b], out_slot(j), ssems.at[b]).wait()

    for b in range(_NBUF):
        g_start(b, b)

    def outer(t, _):
        for b in range(_NBUF):
            j = t * _NBUF + b
            g_wait(j, b)
            s_start(j, b)
            jn = j + _NBUF

            @pl.when(jn < _CHUNKS_PER_W)
            def _():
                s_wait(j, b)
                g_start(jn, b)

        return 0

    lax.fori_loop(0, _CHUNKS_PER_W // _NBUF, outer, 0)
    for b in range(_NBUF):
        j = _CHUNKS_PER_W - _NBUF + b
        s_wait(j, b)


def kernel(token_ids, weight):
    idx = token_ids.reshape(_NW, _CHUNKS_PER_W, _CHUNK).astype(jnp.int32)
    out = _gather_kernel(idx, weight)
    return out.reshape(BATCH, N_FIELDS, EMBEDDING_DIM)


# speedup vs baseline: 1.2977x; 1.0098x over previous
import functools

import jax
import jax.numpy as jnp
from jax import lax
from jax.experimental import pallas as pl
from jax.experimental.pallas import tpu as pltpu
from jax.experimental.pallas import tpu_sc as plsc

NUM_EMBEDDINGS = 100000
EMBEDDING_DIM = 128
BATCH = 4096
N_FIELDS = 26

_NC = 2
_NS = 16
_NW = _NC * _NS

_B = BATCH * N_FIELDS
_CHUNK = 128
_ROWS_PER_W = _B // _NW
_CHUNKS_PER_W = _ROWS_PER_W // _CHUNK
_NBUF = 4
_MAIN = _CHUNKS_PER_W // _NBUF
_REM = _CHUNKS_PER_W - _MAIN * _NBUF


@functools.partial(
    pl.kernel,
    out_type=jax.ShapeDtypeStruct((_B, EMBEDDING_DIM), jnp.float32),
    mesh=plsc.VectorSubcoreMesh(core_axis_name="c", subcore_axis_name="s"),
    scratch_types=[
        pltpu.VMEM((_CHUNKS_PER_W, _CHUNK), jnp.int32),
        pltpu.VMEM((_NBUF, _CHUNK, EMBEDDING_DIM), jnp.float32),
        pltpu.SemaphoreType.DMA((_NBUF,)),
        pltpu.SemaphoreType.DMA((_NBUF,)),
    ],
)
def _gather_kernel(idx_hbm, table_hbm, out_hbm, idx_v, bufs, gsems, ssems):
    wid = lax.axis_index("s") * _NC + lax.axis_index("c")
    pltpu.sync_copy(idx_hbm.at[wid], idx_v)
    row_base = wid * _ROWS_PER_W

    def g_start(j, b):
        pltpu.async_copy(table_hbm.at[idx_v.at[j]], bufs.at[b], gsems.at[b])

    def g_wait(j, b):
        pltpu.make_async_copy(
            table_hbm.at[idx_v.at[j]], bufs.at[b], gsems.at[b]).wait()

    def out_slot(j):
        return out_hbm.at[pl.ds(row_base + j * _CHUNK, _CHUNK)]

    def s_start(j, b):
        pltpu.async_copy(bufs.at[b], out_slot(j), ssems.at[b])

    def s_wait(j, b):
        pltpu.make_async_copy(bufs.at[b], out_slot(j), ssems.at[b]).wait()

    for b in range(_NBUF):
        g_start(b, b)

    def outer(t, _):
        for b in range(_NBUF):
            j = t * _NBUF + b
            g_wait(j, b)
            s_start(j, b)
            jn = j + _NBUF

            @pl.when(jn < _CHUNKS_PER_W)
            def _():
                s_wait(j, b)
                g_start(jn, b)

        return 0

    lax.fori_loop(0, _MAIN, outer, 0)
    for j in range(_MAIN * _NBUF, _CHUNKS_PER_W):
        b = j % _NBUF
        g_wait(j, b)
        s_start(j, b)
    for j in range(_CHUNKS_PER_W - _NBUF, _CHUNKS_PER_W):
        s_wait(j, j % _NBUF)


def kernel(token_ids, weight):
    idx = token_ids.reshape(_NW, _CHUNKS_PER_W, _CHUNK).astype(jnp.int32)
    out = _gather_kernel(idx, weight)
    return out.reshape(BATCH, N_FIELDS, EMBEDDING_DIM)


# speedup vs baseline: 2.0499x; 1.5796x over previous
import functools

import jax
import jax.numpy as jnp
from jax import lax
from jax.experimental import pallas as pl
from jax.experimental.pallas import tpu as pltpu
from jax.experimental.pallas import tpu_sc as plsc

NUM_EMBEDDINGS = 100000
EMBEDDING_DIM = 128
BATCH = 4096
N_FIELDS = 26

_NC = 2
_NS = 16
_NW = _NC * _NS

_ROWS_PER_CHUNK = 4
_CHUNK = _ROWS_PER_CHUNK * N_FIELDS
_BATCH_PER_W = BATCH // _NW
_IDX_PER_W = _BATCH_PER_W * N_FIELDS
_CHUNKS_PER_W = _BATCH_PER_W // _ROWS_PER_CHUNK
_NBUF = 4
_MAIN = _CHUNKS_PER_W // _NBUF


@functools.partial(
    pl.kernel,
    out_type=jax.ShapeDtypeStruct((BATCH, N_FIELDS, EMBEDDING_DIM), jnp.float32),
    mesh=plsc.VectorSubcoreMesh(core_axis_name="c", subcore_axis_name="s"),
    scratch_types=[
        pltpu.VMEM((_IDX_PER_W,), jnp.int32),
        pltpu.VMEM((_NBUF, _CHUNK, EMBEDDING_DIM), jnp.float32),
        pltpu.SemaphoreType.DMA((_NBUF,)),
        pltpu.SemaphoreType.DMA((_NBUF,)),
    ],
)
def _gather_kernel(idx_hbm, table_hbm, out_hbm, idx_v, bufs, gsems, ssems):
    wid = lax.axis_index("s") * _NC + lax.axis_index("c")
    pltpu.sync_copy(idx_hbm.at[wid], idx_v)
    batch_base = wid * _BATCH_PER_W

    def g_start(j, b):
        pltpu.async_copy(
            table_hbm.at[idx_v.at[pl.ds(j * _CHUNK, _CHUNK)]],
            bufs.at[b], gsems.at[b])

    def g_wait(j, b):
        pltpu.make_async_copy(
            table_hbm.at[idx_v.at[pl.ds(j * _CHUNK, _CHUNK)]],
            bufs.at[b], gsems.at[b]).wait()

    def s_descr(j, b, r):
        src = bufs.at[b, pl.ds(r * N_FIELDS, N_FIELDS)]
        dst = out_hbm.at[batch_base + j * _ROWS_PER_CHUNK + r]
        return src, dst

    def s_start(j, b):
        for r in range(_ROWS_PER_CHUNK):
            src, dst = s_descr(j, b, r)
            pltpu.async_copy(src, dst, ssems.at[b])

    def s_wait(j, b):
        for r in range(_ROWS_PER_CHUNK):
            src, dst = s_descr(j, b, r)
            pltpu.make_async_copy(src, dst, ssems.at[b]).wait()

    for b in range(_NBUF):
        g_start(b, b)

    def outer(t, _):
        for b in range(_NBUF):
            j = t * _NBUF + b
            g_wait(j, b)
            s_start(j, b)
            jn = j + _NBUF

            @pl.when(jn < _CHUNKS_PER_W)
            def _():
                s_wait(j, b)
                g_start(jn, b)

        return 0

    lax.fori_loop(0, _MAIN, outer, 0)
    for j in range(_CHUNKS_PER_W - _NBUF, _CHUNKS_PER_W):
        s_wait(j, j % _NBUF)


def kernel(token_ids, weight):
    idx = token_ids.reshape(_NW, _IDX_PER_W).astype(jnp.int32)
    return _gather_kernel(idx, weight)
